# baseline (device time: 187672 ns/iter reference)
import jax
import jax.numpy as jnp
from jax import lax
from jax.experimental import pallas as pl
from jax.experimental.pallas import tpu as pltpu

N_DEV = 8
MESH = pl.DeviceIdType.MESH
N_STREAMS = 4


def kernel(x, w_mat, scale_x, scale_w):
    m, k_per = x.shape
    _, n = w_mat.shape
    mc = m // N_DEV
    qn = n // N_STREAMS

    n_hops = 2 * (N_DEV - 1)
    rs_last = N_DEV - 2

    def ring_id(q):
        return jnp.where(q < 4, q, 11 - q)

    def body(x_ref, w_ref, sx_ref, sw_ref,
             out_ref, bufs, ssend, srecv, scred, osem):
        my = lax.axis_index("i")
        p = ring_id(my)
        right = ring_id(lax.rem(p + 1, N_DEV))
        left = ring_id(lax.rem(p + N_DEV - 1, N_DEV))

        barrier = pltpu.get_barrier_semaphore()
        for nbr in (left, right):
            pl.semaphore_signal(barrier, inc=1, device_id=(nbr,),
                                device_id_type=MESH)
        pl.semaphore_wait(barrier, 2)

        scale = sx_ref[0] * sw_ref[0]

        los = [0, qn, 2 * qn, 3 * qn]
        is_fwd = [True, True, False, False]

        def receiver(k):
            return right if is_fwd[k] else left

        def sender(k):
            return left if is_fwd[k] else right

        def mm2(c, fwd):
            xs = x_ref[pl.ds(c * mc, mc), :].astype(jnp.bfloat16)
            lo = 0 if fwd else 2 * qn
            wb = w_ref[:, lo:lo + 2 * qn].astype(jnp.bfloat16)
            return jnp.dot(xs, wb, preferred_element_type=jnp.float32)

        def parts_for(s):
            pf = mm2(chunk_rs(0, s), True)
            pb = mm2(chunk_rs(2, s), False)
            return [pf[:, :qn], pf[:, qn:], pb[:, :qn], pb[:, qn:]]

        def chunk_rs(k, s):
            if is_fwd[k]:
                return lax.rem(p - s - 1 + 2 * N_DEV, N_DEV)
            return lax.rem(p + s + 1, N_DEV)

        def chunk_ag(k, t):
            if is_fwd[k]:
                return lax.rem(p - t + 2 * N_DEV, N_DEV)
            return lax.rem(p + t, N_DEV)

        def grant(k):
            pl.semaphore_signal(scred.at[k], inc=1, device_id=(sender(k),),
                                device_id_type=MESH)

        def fire(k, h):
            pl.semaphore_wait(scred.at[k], 1)
            r = pltpu.make_async_remote_copy(
                src_ref=bufs.at[k, h % 2], dst_ref=bufs.at[k, (h + 1) % 2],
                send_sem=ssend.at[k], recv_sem=srecv.at[k],
                device_id=(receiver(k),), device_id_type=MESH)
            r.start()
            return r

        def store(k, h, c):
            d = pltpu.make_async_copy(
                bufs.at[k, (h + 1) % 2],
                out_ref.at[pl.ds(c * mc, mc), pl.ds(los[k], qn)],
                osem.at[k])
            d.start()
            return d

        inflight = [None] * N_STREAMS
        own_f = mm2(p, True)
        own_b = mm2(p, False)
        own = [own_f[:, :qn], own_f[:, qn:], own_b[:, :qn], own_b[:, qn:]]
        for k in range(N_STREAMS):
            bufs[k, 0] = own[k].astype(jnp.bfloat16)
            grant(k)
            inflight[k] = fire(k, 0)
        outdma = [None] * N_STREAMS

        parts = parts_for(0)
        for h in range(n_hops):
            for k in range(N_STREAMS):
                part = parts[k]
                inflight[k].wait()
                if outdma[k] is not None:
                    outdma[k].wait()
                    outdma[k] = None
                if h < n_hops - 1:
                    grant(k)
                r = (h + 1) % 2
                if h < rs_last:
                    acc = bufs[k, r].astype(jnp.float32) + part
                    bufs[k, r] = acc.astype(jnp.bfloat16)
                    inflight[k] = fire(k, h + 1)
                elif h == rs_last:
                    acc = bufs[k, r].astype(jnp.float32) + part
                    fin = jnp.maximum(acc * scale, 0.0).astype(jnp.bfloat16)
                    bufs[k, r] = fin
                    inflight[k] = fire(k, h + 1)
                    outdma[k] = store(k, h, chunk_rs(k, h))
                else:
                    if h < n_hops - 1:
                        inflight[k] = fire(k, h + 1)
                    outdma[k] = store(k, h, chunk_ag(k, h - (rs_last + 1)))
            if h < rs_last:
                parts = parts_for(h + 1)

        for k in range(N_STREAMS):
            outdma[k].wait()

    return pl.pallas_call(
        body,
        out_shape=jax.ShapeDtypeStruct((m, n), jnp.bfloat16),
        in_specs=[
            pl.BlockSpec(memory_space=pltpu.VMEM),
            pl.BlockSpec(memory_space=pltpu.VMEM),
            pl.BlockSpec(memory_space=pltpu.SMEM),
            pl.BlockSpec(memory_space=pltpu.SMEM),
        ],
        out_specs=pl.BlockSpec(memory_space=pl.ANY),
        scratch_shapes=[
            pltpu.VMEM((N_STREAMS, 2, mc, qn), jnp.bfloat16),
            pltpu.SemaphoreType.DMA((N_STREAMS,)),
            pltpu.SemaphoreType.DMA((N_STREAMS,)),
            pltpu.SemaphoreType.REGULAR((N_STREAMS,)),
            pltpu.SemaphoreType.DMA((N_STREAMS,)),
        ],
        compiler_params=pltpu.CompilerParams(
            collective_id=0, vmem_limit_bytes=100 * 1024 * 1024),
    )(x, w_mat, scale_x, scale_w)


# device time: 186799 ns/iter; 1.0047x vs baseline; 1.0047x over previous
import jax
import jax.numpy as jnp
from jax import lax
from jax.experimental import pallas as pl
from jax.experimental.pallas import tpu as pltpu

N_DEV = 8
MESH = pl.DeviceIdType.MESH
N_STREAMS = 4


def kernel(x, w_mat, scale_x, scale_w):
    m, k_per = x.shape
    _, n = w_mat.shape
    mc = m // N_DEV
    qn = n // N_STREAMS

    n_hops = 2 * (N_DEV - 1)
    rs_last = N_DEV - 2

    def ring_id(q):
        return jnp.where(q < 4, q, 11 - q)

    def body(x_ref, w_ref, sx_ref, sw_ref,
             out_ref, bufs, ssend, srecv, scred, osem):
        my = lax.axis_index("i")
        p = ring_id(my)
        right = ring_id(lax.rem(p + 1, N_DEV))
        left = ring_id(lax.rem(p + N_DEV - 1, N_DEV))

        barrier = pltpu.get_barrier_semaphore()
        for nbr in (left, right):
            pl.semaphore_signal(barrier, inc=1, device_id=(nbr,),
                                device_id_type=MESH)
        pl.semaphore_wait(barrier, 2)

        scale = sx_ref[0] * sw_ref[0]

        los = [0, qn, 2 * qn, 3 * qn]
        is_fwd = [True, True, False, False]

        def receiver(k):
            return right if is_fwd[k] else left

        def sender(k):
            return left if is_fwd[k] else right

        def mm(c, lo):
            xs = x_ref[pl.ds(c * mc, mc), :].astype(jnp.bfloat16)
            wb = w_ref[:, lo:lo + qn].astype(jnp.bfloat16)
            return jnp.dot(xs, wb, preferred_element_type=jnp.float32)

        def chunk_rs(k, s):
            if is_fwd[k]:
                return lax.rem(p - s - 1 + 2 * N_DEV, N_DEV)
            return lax.rem(p + s + 1, N_DEV)

        def chunk_ag(k, t):
            if is_fwd[k]:
                return lax.rem(p - t + 2 * N_DEV, N_DEV)
            return lax.rem(p + t, N_DEV)

        def grant(k):
            pl.semaphore_signal(scred.at[k], inc=1, device_id=(sender(k),),
                                device_id_type=MESH)

        def fire(k, h):
            pl.semaphore_wait(scred.at[k], 1)
            r = pltpu.make_async_remote_copy(
                src_ref=bufs.at[k, h % 2], dst_ref=bufs.at[k, (h + 1) % 2],
                send_sem=ssend.at[k], recv_sem=srecv.at[k],
                device_id=(receiver(k),), device_id_type=MESH)
            r.start()
            return r

        def store(k, h, c):
            d = pltpu.make_async_copy(
                bufs.at[k, (h + 1) % 2],
                out_ref.at[pl.ds(c * mc, mc), pl.ds(los[k], qn)],
                osem.at[k])
            d.start()
            return d

        inflight = [None] * N_STREAMS
        for k in range(N_STREAMS):
            bufs[k, 0] = mm(p, los[k]).astype(jnp.bfloat16)
            grant(k)
            inflight[k] = fire(k, 0)
        outdma = [None] * N_STREAMS

        for h in range(n_hops):
            parts = [None] * N_STREAMS
            if h <= rs_last:
                for k in range(N_STREAMS):
                    parts[k] = mm(chunk_rs(k, h), los[k])
            for k in range(N_STREAMS):
                lo = los[k]
                part = parts[k]
                inflight[k].wait()
                if outdma[k] is not None:
                    outdma[k].wait()
                    outdma[k] = None
                if h < n_hops - 1:
                    grant(k)
                r = (h + 1) % 2
                if h < rs_last:
                    acc = bufs[k, r].astype(jnp.float32) + part
                    bufs[k, r] = acc.astype(jnp.bfloat16)
                    inflight[k] = fire(k, h + 1)
                elif h == rs_last:
                    acc = bufs[k, r].astype(jnp.float32) + part
                    fin = jnp.maximum(acc * scale, 0.0).astype(jnp.bfloat16)
                    bufs[k, r] = fin
                    inflight[k] = fire(k, h + 1)
                    outdma[k] = store(k, h, chunk_rs(k, h))
                else:
                    if h < n_hops - 1:
                        inflight[k] = fire(k, h + 1)
                    outdma[k] = store(k, h, chunk_ag(k, h - (rs_last + 1)))

        for k in range(N_STREAMS):
            outdma[k].wait()

    return pl.pallas_call(
        body,
        out_shape=jax.ShapeDtypeStruct((m, n), jnp.bfloat16),
        in_specs=[
            pl.BlockSpec(memory_space=pltpu.VMEM),
            pl.BlockSpec(memory_space=pltpu.VMEM),
            pl.BlockSpec(memory_space=pltpu.SMEM),
            pl.BlockSpec(memory_space=pltpu.SMEM),
        ],
        out_specs=pl.BlockSpec(memory_space=pl.ANY),
        scratch_shapes=[
            pltpu.VMEM((N_STREAMS, 2, mc, qn), jnp.bfloat16),
            pltpu.SemaphoreType.DMA((N_STREAMS,)),
            pltpu.SemaphoreType.DMA((N_STREAMS,)),
            pltpu.SemaphoreType.REGULAR((N_STREAMS,)),
            pltpu.SemaphoreType.DMA((N_STREAMS,)),
        ],
        compiler_params=pltpu.CompilerParams(
            collective_id=0, vmem_limit_bytes=100 * 1024 * 1024),
    )(x, w_mat, scale_x, scale_w)
